# baseline (speedup 1.0000x reference)
import jax
import jax.numpy as jnp
from jax import lax
from jax.experimental import pallas as pl
from jax.experimental.pallas import tpu as pltpu
from jax.experimental.pallas import tpu_sc as plsc

NS = 10000
NT = 10000
E = 320000
DE = 16
H = 128
L = 3
B = 32
NPAD = 10240
GROUPS = E // 128
NSUB = 16
GPT = (GROUPS + NSUB - 1) // NSUB
RPT = NPAD // NSUB
FPT = NPAD // NSUB
EAW = NPAD * DE
EAPT = EAW // NSUB
GPT2 = 160
GROUPS2 = GPT2 * NSUB
E2 = GROUPS2 * 128
PADIDX = NS + 100

f32 = jnp.float32

_MESH = plsc.VectorSubcoreMesh(core_axis_name="c", subcore_axis_name="s")


def _deg_body(src3, dst3, zflat, deg_s_out, deg_t_out, acc, idx_v, ones_v):
    c = lax.axis_index("c")
    s = lax.axis_index("s")
    base = s * FPT
    pltpu.sync_copy(zflat.at[pl.ds(base, FPT)], acc.at[pl.ds(base, FPT)])
    for j in range(8):
        ones_v[pl.ds(j * 16, 16)] = jnp.ones((16,), f32)
    plsc.subcore_barrier()

    def body(i, carry):
        g = i * NSUB + s

        @pl.when(g < GROUPS)
        def _():
            @pl.when(c == 0)
            def _():
                pltpu.sync_copy(dst3.at[g], idx_v)

            @pl.when(c == 1)
            def _():
                pltpu.sync_copy(src3.at[g], idx_v)

            pltpu.sync_copy(ones_v, acc.at[idx_v.at[0]], add=True)

        return carry

    lax.fori_loop(0, GPT, body, 0)
    plsc.subcore_barrier()

    @pl.when(c == 0)
    def _():
        pltpu.sync_copy(acc.at[pl.ds(base, FPT)], deg_t_out.at[pl.ds(base, FPT)])

    @pl.when(c == 1)
    def _():
        pltpu.sync_copy(acc.at[pl.ds(base, FPT)], deg_s_out.at[pl.ds(base, FPT)])


_deg_call = pl.kernel(
    _deg_body,
    out_type=(
        jax.ShapeDtypeStruct((NPAD,), f32),
        jax.ShapeDtypeStruct((NPAD,), f32),
    ),
    mesh=_MESH,
    compiler_params=pltpu.CompilerParams(needs_layout_passes=False),
    scratch_types=[
        pltpu.VMEM_SHARED((NPAD,), f32),
        pltpu.VMEM((1, 128), jnp.int32),
        pltpu.VMEM((128,), f32),
    ],
)


def _ea_body(src3, dst3, ea_hbm, invs_flat, invt_flat, zea,
             ea_t_out, ea_s_out,
             acc, widx_v, sidx_v, ea_v, out_v, idx2, wtbl_v, sem):
    c = lax.axis_index("c")
    s = lax.axis_index("s")
    base = s * EAPT
    pltpu.sync_copy(zea.at[pl.ds(base, EAPT)], acc.at[pl.ds(base, EAPT)])

    @pl.when(c == 0)
    def _():
        pltpu.sync_copy(invs_flat, wtbl_v)

    @pl.when(c == 1)
    def _():
        pltpu.sync_copy(invt_flat, wtbl_v)

    plsc.subcore_barrier()
    ii16 = lax.broadcasted_iota(jnp.int32, (16,), 0)

    def group(i, carry):
        g = i * NSUB + s

        @pl.when(g < GROUPS)
        def _():
            @pl.when(c == 0)
            def _():
                pltpu.sync_copy(src3.at[g], widx_v)
                pltpu.sync_copy(dst3.at[g], sidx_v)

            @pl.when(c == 1)
            def _():
                pltpu.sync_copy(dst3.at[g], widx_v)
                pltpu.sync_copy(src3.at[g], sidx_v)

            pltpu.sync_copy(ea_hbm.at[pl.ds(g * 128, 128)], ea_v)

            def sub(j, cc):
                iv = widx_v[0, pl.ds(j * 16, 16)]
                w16 = plsc.load_gather(wtbl_v, [iv])
                dv16 = sidx_v[0, pl.ds(j * 16, 16)]
                for m in range(16):
                    e = j * 16 + m
                    out_v[pl.ds(e * 16, 16)] = ea_v[e, :] * w16[m]
                    idx2[2 * j + (m // 8), pl.ds((m % 8) * 16, 16)] = (
                        dv16[m] * 16 + ii16)
                return cc

            lax.fori_loop(0, 8, sub, 0)
            for r in range(16):
                pltpu.async_copy(out_v.at[pl.ds(r * 128, 128)],
                                 acc.at[idx2.at[r]], sem, add=True)
            for r in range(16):
                pltpu.make_async_copy(out_v.at[pl.ds(r * 128, 128)],
                                      acc.at[idx2.at[r]], sem).wait()

        return carry

    lax.fori_loop(0, GPT, group, 0)
    plsc.subcore_barrier()

    @pl.when(c == 0)
    def _():
        pltpu.sync_copy(acc.at[pl.ds(base, EAPT)], ea_t_out.at[pl.ds(base, EAPT)])

    @pl.when(c == 1)
    def _():
        pltpu.sync_copy(acc.at[pl.ds(base, EAPT)], ea_s_out.at[pl.ds(base, EAPT)])


_ea_call = pl.kernel(
    _ea_body,
    out_type=(
        jax.ShapeDtypeStruct((EAW,), f32),
        jax.ShapeDtypeStruct((EAW,), f32),
    ),
    mesh=_MESH,
    compiler_params=pltpu.CompilerParams(needs_layout_passes=False),
    scratch_types=[
        pltpu.VMEM_SHARED((EAW,), f32),
        pltpu.VMEM((1, 128), jnp.int32),
        pltpu.VMEM((1, 128), jnp.int32),
        pltpu.VMEM((128, DE), f32),
        pltpu.VMEM((128 * DE,), f32),
        pltpu.VMEM((16, 128), jnp.int32),
        pltpu.VMEM((NPAD,), f32),
        pltpu.SemaphoreType.DMA,
    ],
)


def _spmm_body(ps, pt, srcM, dstM, z128, acc_t_out, acc_s_out,
               acc, gblk, sblk, rows0, rows1, sem0, sem1):
    c = lax.axis_index("c")
    s = lax.axis_index("s")
    base = s * RPT
    pltpu.sync_copy(z128.at[pl.ds(base, RPT)], acc.at[pl.ds(base, RPT)])
    plsc.subcore_barrier()

    rows = (rows0, rows1)
    sems = (sem0, sem1)

    def fire(j, p):
        @pl.when(c == 0)
        def _():
            pltpu.async_copy(ps.at[gblk.at[j]], rows[p], sems[p])

        @pl.when(c == 1)
        def _():
            pltpu.async_copy(pt.at[gblk.at[j]], rows[p], sems[p])

    def scat(j, p):
        @pl.when(c == 0)
        def _():
            pltpu.make_async_copy(ps.at[gblk.at[j]], rows[p], sems[p]).wait()

        @pl.when(c == 1)
        def _():
            pltpu.make_async_copy(pt.at[gblk.at[j]], rows[p], sems[p]).wait()

        pltpu.sync_copy(rows[p], acc.at[sblk.at[j]], add=True)

    def blk_body(i2, carry):
        base_g = s * GPT2 + i2 * 16

        @pl.when(c == 0)
        def _():
            pltpu.sync_copy(srcM.at[pl.ds(base_g, 16)], gblk)
            pltpu.sync_copy(dstM.at[pl.ds(base_g, 16)], sblk)

        @pl.when(c == 1)
        def _():
            pltpu.sync_copy(dstM.at[pl.ds(base_g, 16)], gblk)
            pltpu.sync_copy(srcM.at[pl.ds(base_g, 16)], sblk)

        fire(0, 0)
        for j in range(16):
            p = j & 1
            if j + 1 < 16:
                fire(j + 1, 1 - p)
            scat(j, p)
        return carry

    lax.fori_loop(0, GPT2 // 16, blk_body, 0)
    plsc.subcore_barrier()

    @pl.when(c == 0)
    def _():
        pltpu.sync_copy(acc.at[pl.ds(base, RPT)], acc_t_out.at[pl.ds(base, RPT)])

    @pl.when(c == 1)
    def _():
        pltpu.sync_copy(acc.at[pl.ds(base, RPT)], acc_s_out.at[pl.ds(base, RPT)])


_spmm_call = pl.kernel(
    _spmm_body,
    out_type=(
        jax.ShapeDtypeStruct((NPAD, H), f32),
        jax.ShapeDtypeStruct((NPAD, H), f32),
    ),
    mesh=_MESH,
    compiler_params=pltpu.CompilerParams(needs_layout_passes=False),
    scratch_types=[
        pltpu.VMEM_SHARED((NPAD, H), f32),
        pltpu.VMEM((16, 128), jnp.int32),
        pltpu.VMEM((16, 128), jnp.int32),
        pltpu.VMEM((128, H), f32),
        pltpu.VMEM((128, H), f32),
        pltpu.SemaphoreType.DMA,
        pltpu.SemaphoreType.DMA,
    ],
)


_NB = 10
_BR = NPAD // _NB


def _inv_body(ds_ref, dt_ref, is_ref, it_ref):
    is_ref[...] = lax.rsqrt(jnp.maximum(ds_ref[...], 1.0))
    it_ref[...] = lax.rsqrt(jnp.maximum(dt_ref[...], 1.0))


def _inv_call(deg_s80, deg_t80):
    return pl.pallas_call(
        _inv_body,
        out_shape=(
            jax.ShapeDtypeStruct((NPAD // 128, 128), f32),
            jax.ShapeDtypeStruct((NPAD // 128, 128), f32),
        ),
    )(deg_s80, deg_t80)


def _t0_body(xs, ds, Ws0, bs0, xt, dt, Wt0, bt0, Ps, Pt):
    inv_s = lax.rsqrt(jnp.maximum(ds[...], 1.0))
    inv_t = lax.rsqrt(jnp.maximum(dt[...], 1.0))
    Ps[...] = (jnp.dot(xs[...], Ws0[...], preferred_element_type=f32)
               + bs0[...]) * inv_s
    Pt[...] = (jnp.dot(xt[...], Wt0[...], preferred_element_type=f32)
               + bt0[...]) * inv_t


def _row_spec(w):
    return pl.BlockSpec((_BR, w), lambda i: (i, 0))


def _full_spec(r, c):
    return pl.BlockSpec((r, c), lambda i: (0, 0))


def _t0_call(x_s, degcol_s, Ws0, bs0, x_t, degcol_t, Wt0, bt0):
    return pl.pallas_call(
        _t0_body,
        grid=(_NB,),
        in_specs=[
            _row_spec(H), _row_spec(1), _full_spec(H, H), _full_spec(1, H),
            _row_spec(H), _row_spec(1), _full_spec(H, H), _full_spec(1, H),
        ],
        out_specs=[_row_spec(H), _row_spec(H)],
        out_shape=(
            jax.ShapeDtypeStruct((NPAD, H), f32),
            jax.ShapeDtypeStruct((NPAD, H), f32),
        ),
    )(x_s, degcol_s, Ws0, bs0, x_t, degcol_t, Wt0, bt0)


def _t_body(accS, eaS, WetP, Wsl, bsl, ds,
            accT, eaT, WesP, Wtl, btl, dt,
            hs_o, Ps_o, ht_o, Pt_o):
    inv_s = lax.rsqrt(jnp.maximum(ds[...], 1.0))
    inv_t = lax.rsqrt(jnp.maximum(dt[...], 1.0))
    hs = jnp.maximum(inv_s * (accS[...] + jnp.dot(
        eaS[...], WetP[...], preferred_element_type=f32)), 0.0)
    ht = jnp.maximum(inv_t * (accT[...] + jnp.dot(
        eaT[...], WesP[...], preferred_element_type=f32)), 0.0)
    hs_o[...] = hs
    ht_o[...] = ht
    Ps_o[...] = (jnp.dot(hs, Wsl[...], preferred_element_type=f32)
                 + bsl[...]) * inv_s
    Pt_o[...] = (jnp.dot(ht, Wtl[...], preferred_element_type=f32)
                 + btl[...]) * inv_t


def _t_call(accS, eaS, WetP, Wsl, bsl, degcol_s,
            accT, eaT, WesP, Wtl, btl, degcol_t):
    return pl.pallas_call(
        _t_body,
        grid=(_NB,),
        in_specs=[
            _row_spec(H), _row_spec(DE), _full_spec(DE, H),
            _full_spec(H, H), _full_spec(1, H), _row_spec(1),
            _row_spec(H), _row_spec(DE), _full_spec(DE, H),
            _full_spec(H, H), _full_spec(1, H), _row_spec(1),
        ],
        out_specs=[_row_spec(H), _row_spec(H), _row_spec(H), _row_spec(H)],
        out_shape=(
            jax.ShapeDtypeStruct((NPAD, H), f32),
            jax.ShapeDtypeStruct((NPAD, H), f32),
            jax.ShapeDtypeStruct((NPAD, H), f32),
            jax.ShapeDtypeStruct((NPAD, H), f32),
        ),
    )(accS, eaS, WetP, Wsl, bsl, degcol_s,
      accT, eaT, WesP, Wtl, btl, degcol_t)


def _f_body(accS, eaS, WetP, ds, hs1, hs2, ids_s,
            accT, eaT, WesP, dt, ht1, ht2, ids_t,
            W1a, b1a, W2a, b2a, W1b, b1b, W2b, b2b, Wm1, bm1, Wm2, bm2,
            out, gs_ref, gt_ref):
    i = pl.program_id(0)

    @pl.when(i == 0)
    def _():
        gs_ref[...] = jnp.zeros_like(gs_ref)
        gt_ref[...] = jnp.zeros_like(gt_ref)

    inv_s = lax.rsqrt(jnp.maximum(ds[...], 1.0))
    inv_t = lax.rsqrt(jnp.maximum(dt[...], 1.0))
    hs3 = jnp.maximum(inv_s * (accS[...] + jnp.dot(
        eaS[...], WetP[...], preferred_element_type=f32)), 0.0)
    ht3 = jnp.maximum(inv_t * (accT[...] + jnp.dot(
        eaT[...], WesP[...], preferred_element_type=f32)), 0.0)
    hcat_s = jnp.concatenate([hs1[...], hs2[...], hs3], axis=1)
    hcat_t = jnp.concatenate([ht1[...], ht2[...], ht3], axis=1)
    oh_s = (lax.broadcasted_iota(jnp.int32, (B, _BR), 0)
            == ids_s[0, 0, :][None, :]).astype(f32)
    oh_t = (lax.broadcasted_iota(jnp.int32, (B, _BR), 0)
            == ids_t[0, 0, :][None, :]).astype(f32)
    gs_ref[...] += jnp.dot(oh_s, hcat_s, preferred_element_type=f32)
    gt_ref[...] += jnp.dot(oh_t, hcat_t, preferred_element_type=f32)

    @pl.when(i == _NB - 1)
    def _():
        x = jnp.concatenate([gs_ref[...], gt_ref[...]], axis=1)
        h1 = jnp.maximum(jnp.dot(x, W1a[...], preferred_element_type=f32)
                         + b1a[...], 0.0)
        x1 = jnp.dot(h1, W2a[...], preferred_element_type=f32) + b2a[...]
        h2 = jnp.maximum(jnp.dot(x, W1b[...], preferred_element_type=f32)
                         + b1b[...], 0.0)
        x2 = jnp.dot(h2, W2b[...], preferred_element_type=f32) + b2b[...]
        x12 = jnp.concatenate([x1, x2], axis=1)
        hm = jnp.maximum(jnp.dot(x12, Wm1[...], preferred_element_type=f32)
                         + bm1[...], 0.0)
        out[...] = jnp.dot(hm, Wm2[...], preferred_element_type=f32) + bm2[...]


def _f_call(accS, eaS, WetP, degcol_s, hs1, hs2, ids_s3,
            accT, eaT, WesP, degcol_t, ht1, ht2, ids_t3,
            W1a, b1a, W2a, b2a, W1b, b1b, W2b, b2b, Wm1, bm1, Wm2, bm2):
    ids_spec = pl.BlockSpec((1, 1, _BR), lambda i: (i, 0, 0))
    return pl.pallas_call(
        _f_body,
        grid=(_NB,),
        in_specs=[
            _row_spec(H), _row_spec(DE), _full_spec(DE, H), _row_spec(1),
            _row_spec(H), _row_spec(H), ids_spec,
            _row_spec(H), _row_spec(DE), _full_spec(DE, H), _row_spec(1),
            _row_spec(H), _row_spec(H), ids_spec,
            _full_spec(2 * L * H, H), _full_spec(1, H),
            _full_spec(H, 5), _full_spec(1, 5),
            _full_spec(2 * L * H, H), _full_spec(1, H),
            _full_spec(H, 5), _full_spec(1, 5),
            _full_spec(10, H), _full_spec(1, H),
            _full_spec(H, 1), _full_spec(1, 1),
        ],
        out_specs=pl.BlockSpec((B, 1), lambda i: (0, 0)),
        out_shape=jax.ShapeDtypeStruct((B, 1), f32),
        scratch_shapes=[
            pltpu.VMEM((B, L * H), f32),
            pltpu.VMEM((B, L * H), f32),
        ],
    )(accS, eaS, WetP, degcol_s, hs1, hs2, ids_s3,
      accT, eaT, WesP, degcol_t, ht1, ht2, ids_t3,
      W1a, b1a, W2a, b2a, W1b, b1b, W2b, b2b, Wm1, bm1, Wm2, bm2)


def kernel(x_s, x_t, edge_attr, edge_index, x_s_batch, x_t_batch,
           Ws, Wt, Wes, Wet, bs, bt,
           W1a, b1a, W2a, b2a, W1b, b1b, W2b, b2b,
           Wm1, bm1, Wm2, bm2):
    src_i = edge_index[0].astype(jnp.int32)
    dst_i = edge_index[1].astype(jnp.int32)
    src3 = src_i.reshape(GROUPS, 1, 128)
    dst3 = dst_i.reshape(GROUPS, 1, 128)
    pad_i = jnp.full((E2 - E,), PADIDX, jnp.int32)
    srcM = jnp.concatenate([src_i, pad_i]).reshape(GROUPS2, 128)
    dstM = jnp.concatenate([dst_i, pad_i]).reshape(GROUPS2, 128)
    zflat = jnp.zeros((NPAD,), f32)
    zea = jnp.zeros((EAW,), f32)
    z128 = jnp.zeros((NPAD, H), f32)
    zrow = jnp.zeros((NPAD - NS, H), f32)
    x_sp = jnp.concatenate([x_s, zrow])
    x_tp = jnp.concatenate([x_t, zrow])

    deg_s, deg_t = _deg_call(src3, dst3, zflat)
    inv_s80, inv_t80 = _inv_call(deg_s.reshape(NPAD // 128, 128),
                                 deg_t.reshape(NPAD // 128, 128))
    invs_flat = inv_s80.reshape(NPAD)
    invt_flat = inv_t80.reshape(NPAD)
    degcol_s = deg_s.reshape(NPAD, 1)
    degcol_t = deg_t.reshape(NPAD, 1)

    ea_tp, ea_sp = _ea_call(src3, dst3, edge_attr, invs_flat, invt_flat, zea)
    ea_t = ea_tp.reshape(NPAD, DE)
    ea_s = ea_sp.reshape(NPAD, DE)

    bs_r = bs.reshape(L, 1, H)
    bt_r = bt.reshape(L, 1, H)

    Ps, Pt = _t0_call(x_sp, degcol_s, Ws[0], bs_r[0], x_tp, degcol_t, Wt[0], bt_r[0])
    accT, accS = _spmm_call(Ps, Pt, srcM, dstM, z128)
    hs1, Ps, ht1, Pt = _t_call(accS, ea_s, Wet[0], Ws[1], bs_r[1], degcol_s,
                               accT, ea_t, Wes[0], Wt[1], bt_r[1], degcol_t)
    accT, accS = _spmm_call(Ps, Pt, srcM, dstM, z128)
    hs2, Ps, ht2, Pt = _t_call(accS, ea_s, Wet[1], Ws[2], bs_r[2], degcol_s,
                               accT, ea_t, Wes[1], Wt[2], bt_r[2], degcol_t)
    accT, accS = _spmm_call(Ps, Pt, srcM, dstM, z128)

    pad_b = jnp.full((NPAD - NS,), B, jnp.int32)
    ids_s3 = jnp.concatenate([x_s_batch.astype(jnp.int32), pad_b]).reshape(
        _NB, 1, _BR)
    ids_t3 = jnp.concatenate([x_t_batch.astype(jnp.int32), pad_b]).reshape(
        _NB, 1, _BR)
    out = _f_call(accS, ea_s, Wet[2], degcol_s, hs1, hs2, ids_s3,
                  accT, ea_t, Wes[2], degcol_t, ht1, ht2, ids_t3,
                  W1a, b1a.reshape(1, -1), W2a, b2a.reshape(1, -1),
                  W1b, b1b.reshape(1, -1), W2b, b2b.reshape(1, -1),
                  Wm1, bm1.reshape(1, -1), Wm2, bm2.reshape(1, -1))
    return out


# speedup vs baseline: 1.7746x; 1.7746x over previous
import jax
import jax.numpy as jnp
from jax import lax
from jax.experimental import pallas as pl
from jax.experimental.pallas import tpu as pltpu
from jax.experimental.pallas import tpu_sc as plsc

NS = 10000
NT = 10000
E = 320000
DE = 16
H = 128
L = 3
B = 32
NPAD = 10240
GROUPS = E // 128
NSUB = 16
GPT = (GROUPS + NSUB - 1) // NSUB
RPT = NPAD // NSUB
FPT = NPAD // NSUB
EAW = NPAD * DE
EAPT = EAW // NSUB
GPT2 = 160
GROUPS2 = GPT2 * NSUB
E2 = GROUPS2 * 128
PADIDX = NS + 100

f32 = jnp.float32

_MESH = plsc.VectorSubcoreMesh(core_axis_name="c", subcore_axis_name="s")


def _deg_body(src3, dst3, zflat, deg_s_out, deg_t_out, acc, idx_v, ones_v):
    c = lax.axis_index("c")
    s = lax.axis_index("s")
    base = s * FPT
    pltpu.sync_copy(zflat.at[pl.ds(base, FPT)], acc.at[pl.ds(base, FPT)])
    for j in range(8):
        ones_v[pl.ds(j * 16, 16)] = jnp.ones((16,), f32)
    plsc.subcore_barrier()

    def body(i, carry):
        g = i * NSUB + s

        @pl.when(g < GROUPS)
        def _():
            @pl.when(c == 0)
            def _():
                pltpu.sync_copy(dst3.at[g], idx_v)

            @pl.when(c == 1)
            def _():
                pltpu.sync_copy(src3.at[g], idx_v)

            pltpu.sync_copy(ones_v, acc.at[idx_v.at[0]], add=True)

        return carry

    lax.fori_loop(0, GPT, body, 0)
    plsc.subcore_barrier()

    @pl.when(c == 0)
    def _():
        pltpu.sync_copy(acc.at[pl.ds(base, FPT)], deg_t_out.at[pl.ds(base, FPT)])

    @pl.when(c == 1)
    def _():
        pltpu.sync_copy(acc.at[pl.ds(base, FPT)], deg_s_out.at[pl.ds(base, FPT)])


_deg_call = pl.kernel(
    _deg_body,
    out_type=(
        jax.ShapeDtypeStruct((NPAD,), f32),
        jax.ShapeDtypeStruct((NPAD,), f32),
    ),
    mesh=_MESH,
    compiler_params=pltpu.CompilerParams(needs_layout_passes=False),
    scratch_types=[
        pltpu.VMEM_SHARED((NPAD,), f32),
        pltpu.VMEM((1, 128), jnp.int32),
        pltpu.VMEM((128,), f32),
    ],
)


def _ea_body(src3, dst3, ea_hbm, invs_flat, invt_flat, zea,
             ea_t_out, ea_s_out,
             acc, widx_v, sidx_v, ea_v, out_v, idx2, wtbl_v, sem):
    c = lax.axis_index("c")
    s = lax.axis_index("s")
    base = s * EAPT
    pltpu.sync_copy(zea.at[pl.ds(base, EAPT)], acc.at[pl.ds(base, EAPT)])

    @pl.when(c == 0)
    def _():
        pltpu.sync_copy(invs_flat, wtbl_v)

    @pl.when(c == 1)
    def _():
        pltpu.sync_copy(invt_flat, wtbl_v)

    plsc.subcore_barrier()
    ii16 = lax.broadcasted_iota(jnp.int32, (16,), 0)

    def group(i, carry):
        g = i * NSUB + s

        @pl.when(g < GROUPS)
        def _():
            @pl.when(c == 0)
            def _():
                pltpu.sync_copy(src3.at[g], widx_v)
                pltpu.sync_copy(dst3.at[g], sidx_v)

            @pl.when(c == 1)
            def _():
                pltpu.sync_copy(dst3.at[g], widx_v)
                pltpu.sync_copy(src3.at[g], sidx_v)

            pltpu.sync_copy(ea_hbm.at[pl.ds(g * 128, 128)], ea_v)

            def sub(j, cc):
                iv = widx_v[0, pl.ds(j * 16, 16)]
                w16 = plsc.load_gather(wtbl_v, [iv])
                dv16 = sidx_v[0, pl.ds(j * 16, 16)]
                for m in range(16):
                    e = j * 16 + m
                    out_v[pl.ds(e * 16, 16)] = ea_v[e, :] * w16[m]
                    idx2[2 * j + (m // 8), pl.ds((m % 8) * 16, 16)] = (
                        dv16[m] * 16 + ii16)
                return cc

            lax.fori_loop(0, 8, sub, 0)
            for r in range(16):
                pltpu.async_copy(out_v.at[pl.ds(r * 128, 128)],
                                 acc.at[idx2.at[r]], sem, add=True)
            for r in range(16):
                pltpu.make_async_copy(out_v.at[pl.ds(r * 128, 128)],
                                      acc.at[idx2.at[r]], sem).wait()

        return carry

    lax.fori_loop(0, GPT, group, 0)
    plsc.subcore_barrier()

    @pl.when(c == 0)
    def _():
        pltpu.sync_copy(acc.at[pl.ds(base, EAPT)], ea_t_out.at[pl.ds(base, EAPT)])

    @pl.when(c == 1)
    def _():
        pltpu.sync_copy(acc.at[pl.ds(base, EAPT)], ea_s_out.at[pl.ds(base, EAPT)])


_ea_call = pl.kernel(
    _ea_body,
    out_type=(
        jax.ShapeDtypeStruct((EAW,), f32),
        jax.ShapeDtypeStruct((EAW,), f32),
    ),
    mesh=_MESH,
    compiler_params=pltpu.CompilerParams(needs_layout_passes=False),
    scratch_types=[
        pltpu.VMEM_SHARED((EAW,), f32),
        pltpu.VMEM((1, 128), jnp.int32),
        pltpu.VMEM((1, 128), jnp.int32),
        pltpu.VMEM((128, DE), f32),
        pltpu.VMEM((128 * DE,), f32),
        pltpu.VMEM((16, 128), jnp.int32),
        pltpu.VMEM((NPAD,), f32),
        pltpu.SemaphoreType.DMA,
    ],
)


def _spmm_body(ps, pt, srcM, dstM, z128, acc_t_out, acc_s_out,
               acc, gblk, sblk, rows0, rows1, sem0, sem1):
    c = lax.axis_index("c")
    s = lax.axis_index("s")
    base = s * RPT
    pltpu.sync_copy(z128.at[pl.ds(base, RPT)], acc.at[pl.ds(base, RPT)])
    plsc.subcore_barrier()

    rows = (rows0, rows1)
    sems = (sem0, sem1)

    def fire(j, p):
        @pl.when(c == 0)
        def _():
            pltpu.async_copy(ps.at[gblk.at[j]], rows[p], sems[p])

        @pl.when(c == 1)
        def _():
            pltpu.async_copy(pt.at[gblk.at[j]], rows[p], sems[p])

    def scat(j, p):
        @pl.when(c == 0)
        def _():
            pltpu.make_async_copy(ps.at[gblk.at[j]], rows[p], sems[p]).wait()

        @pl.when(c == 1)
        def _():
            pltpu.make_async_copy(pt.at[gblk.at[j]], rows[p], sems[p]).wait()

        pltpu.sync_copy(rows[p], acc.at[sblk.at[j]], add=True)

    def blk_body(i2, carry):
        base_g = s * GPT2 + i2 * 16

        @pl.when(c == 0)
        def _():
            pltpu.sync_copy(srcM.at[pl.ds(base_g, 16)], gblk)
            pltpu.sync_copy(dstM.at[pl.ds(base_g, 16)], sblk)

        @pl.when(c == 1)
        def _():
            pltpu.sync_copy(dstM.at[pl.ds(base_g, 16)], gblk)
            pltpu.sync_copy(srcM.at[pl.ds(base_g, 16)], sblk)

        fire(0, 0)
        for j in range(16):
            p = j & 1
            if j + 1 < 16:
                fire(j + 1, 1 - p)
            scat(j, p)
        return carry

    lax.fori_loop(0, GPT2 // 16, blk_body, 0)
    plsc.subcore_barrier()

    @pl.when(c == 0)
    def _():
        pltpu.sync_copy(acc.at[pl.ds(base, RPT)], acc_t_out.at[pl.ds(base, RPT)])

    @pl.when(c == 1)
    def _():
        pltpu.sync_copy(acc.at[pl.ds(base, RPT)], acc_s_out.at[pl.ds(base, RPT)])


_spmm_call = pl.kernel(
    _spmm_body,
    out_type=(
        jax.ShapeDtypeStruct((NPAD, H), f32),
        jax.ShapeDtypeStruct((NPAD, H), f32),
    ),
    mesh=_MESH,
    compiler_params=pltpu.CompilerParams(needs_layout_passes=False),
    scratch_types=[
        pltpu.VMEM_SHARED((NPAD, H), f32),
        pltpu.VMEM((16, 128), jnp.int32),
        pltpu.VMEM((16, 128), jnp.int32),
        pltpu.VMEM((128, H), f32),
        pltpu.VMEM((128, H), f32),
        pltpu.SemaphoreType.DMA,
        pltpu.SemaphoreType.DMA,
    ],
)


_NB = 10
_BR = NPAD // _NB


def _inv_body(ds_ref, dt_ref, is_ref, it_ref):
    is_ref[...] = lax.rsqrt(jnp.maximum(ds_ref[...], 1.0))
    it_ref[...] = lax.rsqrt(jnp.maximum(dt_ref[...], 1.0))


def _inv_call(deg_s80, deg_t80):
    return pl.pallas_call(
        _inv_body,
        out_shape=(
            jax.ShapeDtypeStruct((NPAD // 128, 128), f32),
            jax.ShapeDtypeStruct((NPAD // 128, 128), f32),
        ),
    )(deg_s80, deg_t80)


def _t0_body(xs, ds, Ws0, bs0, xt, dt, Wt0, bt0, Ps, Pt):
    inv_s = lax.rsqrt(jnp.maximum(ds[...], 1.0))
    inv_t = lax.rsqrt(jnp.maximum(dt[...], 1.0))
    Ps[...] = (jnp.dot(xs[...], Ws0[...], preferred_element_type=f32)
               + bs0[...]) * inv_s
    Pt[...] = (jnp.dot(xt[...], Wt0[...], preferred_element_type=f32)
               + bt0[...]) * inv_t


def _row_spec(w):
    return pl.BlockSpec((_BR, w), lambda i: (i, 0))


def _full_spec(r, c):
    return pl.BlockSpec((r, c), lambda i: (0, 0))


def _t0_call(x_s, degcol_s, Ws0, bs0, x_t, degcol_t, Wt0, bt0):
    return pl.pallas_call(
        _t0_body,
        grid=(_NB,),
        in_specs=[
            _row_spec(H), _row_spec(1), _full_spec(H, H), _full_spec(1, H),
            _row_spec(H), _row_spec(1), _full_spec(H, H), _full_spec(1, H),
        ],
        out_specs=[_row_spec(H), _row_spec(H)],
        out_shape=(
            jax.ShapeDtypeStruct((NPAD, H), f32),
            jax.ShapeDtypeStruct((NPAD, H), f32),
        ),
    )(x_s, degcol_s, Ws0, bs0, x_t, degcol_t, Wt0, bt0)


def _t_body(accS, eaS, WetP, Wsl, bsl, ds,
            accT, eaT, WesP, Wtl, btl, dt,
            hs_o, Ps_o, ht_o, Pt_o):
    inv_s = lax.rsqrt(jnp.maximum(ds[...], 1.0))
    inv_t = lax.rsqrt(jnp.maximum(dt[...], 1.0))
    hs = jnp.maximum(inv_s * (accS[...] + jnp.dot(
        eaS[...], WetP[...], preferred_element_type=f32)), 0.0)
    ht = jnp.maximum(inv_t * (accT[...] + jnp.dot(
        eaT[...], WesP[...], preferred_element_type=f32)), 0.0)
    hs_o[...] = hs
    ht_o[...] = ht
    Ps_o[...] = (jnp.dot(hs, Wsl[...], preferred_element_type=f32)
                 + bsl[...]) * inv_s
    Pt_o[...] = (jnp.dot(ht, Wtl[...], preferred_element_type=f32)
                 + btl[...]) * inv_t


def _t_call(accS, eaS, WetP, Wsl, bsl, degcol_s,
            accT, eaT, WesP, Wtl, btl, degcol_t):
    return pl.pallas_call(
        _t_body,
        grid=(_NB,),
        in_specs=[
            _row_spec(H), _row_spec(DE), _full_spec(DE, H),
            _full_spec(H, H), _full_spec(1, H), _row_spec(1),
            _row_spec(H), _row_spec(DE), _full_spec(DE, H),
            _full_spec(H, H), _full_spec(1, H), _row_spec(1),
        ],
        out_specs=[_row_spec(H), _row_spec(H), _row_spec(H), _row_spec(H)],
        out_shape=(
            jax.ShapeDtypeStruct((NPAD, H), f32),
            jax.ShapeDtypeStruct((NPAD, H), f32),
            jax.ShapeDtypeStruct((NPAD, H), f32),
            jax.ShapeDtypeStruct((NPAD, H), f32),
        ),
    )(accS, eaS, WetP, Wsl, bsl, degcol_s,
      accT, eaT, WesP, Wtl, btl, degcol_t)


def _f_body(accS, eaS, WetP, ds, hs1, hs2, ids_s,
            accT, eaT, WesP, dt, ht1, ht2, ids_t,
            W1a, b1a, W2a, b2a, W1b, b1b, W2b, b2b, Wm1, bm1, Wm2, bm2,
            out, gs_ref, gt_ref):
    i = pl.program_id(0)

    @pl.when(i == 0)
    def _():
        gs_ref[...] = jnp.zeros_like(gs_ref)
        gt_ref[...] = jnp.zeros_like(gt_ref)

    inv_s = lax.rsqrt(jnp.maximum(ds[...], 1.0))
    inv_t = lax.rsqrt(jnp.maximum(dt[...], 1.0))
    hs3 = jnp.maximum(inv_s * (accS[...] + jnp.dot(
        eaS[...], WetP[...], preferred_element_type=f32)), 0.0)
    ht3 = jnp.maximum(inv_t * (accT[...] + jnp.dot(
        eaT[...], WesP[...], preferred_element_type=f32)), 0.0)
    hcat_s = jnp.concatenate([hs1[...], hs2[...], hs3], axis=1)
    hcat_t = jnp.concatenate([ht1[...], ht2[...], ht3], axis=1)
    oh_s = (lax.broadcasted_iota(jnp.int32, (B, _BR), 0)
            == ids_s[0, 0, :][None, :]).astype(f32)
    oh_t = (lax.broadcasted_iota(jnp.int32, (B, _BR), 0)
            == ids_t[0, 0, :][None, :]).astype(f32)
    gs_ref[...] += jnp.dot(oh_s, hcat_s, preferred_element_type=f32)
    gt_ref[...] += jnp.dot(oh_t, hcat_t, preferred_element_type=f32)

    @pl.when(i == _NB - 1)
    def _():
        x = jnp.concatenate([gs_ref[...], gt_ref[...]], axis=1)
        h1 = jnp.maximum(jnp.dot(x, W1a[...], preferred_element_type=f32)
                         + b1a[...], 0.0)
        x1 = jnp.dot(h1, W2a[...], preferred_element_type=f32) + b2a[...]
        h2 = jnp.maximum(jnp.dot(x, W1b[...], preferred_element_type=f32)
                         + b1b[...], 0.0)
        x2 = jnp.dot(h2, W2b[...], preferred_element_type=f32) + b2b[...]
        x12 = jnp.concatenate([x1, x2], axis=1)
        hm = jnp.maximum(jnp.dot(x12, Wm1[...], preferred_element_type=f32)
                         + bm1[...], 0.0)
        out[...] = jnp.dot(hm, Wm2[...], preferred_element_type=f32) + bm2[...]


def _f_call(accS, eaS, WetP, degcol_s, hs1, hs2, ids_s3,
            accT, eaT, WesP, degcol_t, ht1, ht2, ids_t3,
            W1a, b1a, W2a, b2a, W1b, b1b, W2b, b2b, Wm1, bm1, Wm2, bm2):
    ids_spec = pl.BlockSpec((1, 1, _BR), lambda i: (i, 0, 0))
    return pl.pallas_call(
        _f_body,
        grid=(_NB,),
        in_specs=[
            _row_spec(H), _row_spec(DE), _full_spec(DE, H), _row_spec(1),
            _row_spec(H), _row_spec(H), ids_spec,
            _row_spec(H), _row_spec(DE), _full_spec(DE, H), _row_spec(1),
            _row_spec(H), _row_spec(H), ids_spec,
            _full_spec(2 * L * H, H), _full_spec(1, H),
            _full_spec(H, 5), _full_spec(1, 5),
            _full_spec(2 * L * H, H), _full_spec(1, H),
            _full_spec(H, 5), _full_spec(1, 5),
            _full_spec(10, H), _full_spec(1, H),
            _full_spec(H, 1), _full_spec(1, 1),
        ],
        out_specs=pl.BlockSpec((B, 1), lambda i: (0, 0)),
        out_shape=jax.ShapeDtypeStruct((B, 1), f32),
        scratch_shapes=[
            pltpu.VMEM((B, L * H), f32),
            pltpu.VMEM((B, L * H), f32),
        ],
    )(accS, eaS, WetP, degcol_s, hs1, hs2, ids_s3,
      accT, eaT, WesP, degcol_t, ht1, ht2, ids_t3,
      W1a, b1a, W2a, b2a, W1b, b1b, W2b, b2b, Wm1, bm1, Wm2, bm2)


def kernel(x_s, x_t, edge_attr, edge_index, x_s_batch, x_t_batch,
           Ws, Wt, Wes, Wet, bs, bt,
           W1a, b1a, W2a, b2a, W1b, b1b, W2b, b2b,
           Wm1, bm1, Wm2, bm2):
    src_i = edge_index[0].astype(jnp.int32)
    dst_i = edge_index[1].astype(jnp.int32)
    src3 = src_i.reshape(GROUPS, 1, 128)
    dst3 = dst_i.reshape(GROUPS, 1, 128)
    pad_i = NS + jnp.arange(E2 - E, dtype=jnp.int32) % (NPAD - NS)
    srcM = jnp.concatenate([src_i, pad_i]).reshape(GROUPS2, 128)
    dstM = jnp.concatenate([dst_i, pad_i]).reshape(GROUPS2, 128)
    zflat = jnp.zeros((NPAD,), f32)
    zea = jnp.zeros((EAW,), f32)
    z128 = jnp.zeros((NPAD, H), f32)
    zrow = jnp.zeros((NPAD - NS, H), f32)
    x_sp = jnp.concatenate([x_s, zrow])
    x_tp = jnp.concatenate([x_t, zrow])

    deg_s, deg_t = _deg_call(src3, dst3, zflat)
    inv_s80, inv_t80 = _inv_call(deg_s.reshape(NPAD // 128, 128),
                                 deg_t.reshape(NPAD // 128, 128))
    invs_flat = inv_s80.reshape(NPAD)
    invt_flat = inv_t80.reshape(NPAD)
    degcol_s = deg_s.reshape(NPAD, 1)
    degcol_t = deg_t.reshape(NPAD, 1)

    ea_tp, ea_sp = _ea_call(src3, dst3, edge_attr, invs_flat, invt_flat, zea)
    ea_t = ea_tp.reshape(NPAD, DE)
    ea_s = ea_sp.reshape(NPAD, DE)

    bs_r = bs.reshape(L, 1, H)
    bt_r = bt.reshape(L, 1, H)

    Ps, Pt = _t0_call(x_sp, degcol_s, Ws[0], bs_r[0], x_tp, degcol_t, Wt[0], bt_r[0])
    accT, accS = _spmm_call(Ps, Pt, srcM, dstM, z128)
    hs1, Ps, ht1, Pt = _t_call(accS, ea_s, Wet[0], Ws[1], bs_r[1], degcol_s,
                               accT, ea_t, Wes[0], Wt[1], bt_r[1], degcol_t)
    accT, accS = _spmm_call(Ps, Pt, srcM, dstM, z128)
    hs2, Ps, ht2, Pt = _t_call(accS, ea_s, Wet[1], Ws[2], bs_r[2], degcol_s,
                               accT, ea_t, Wes[1], Wt[2], bt_r[2], degcol_t)
    accT, accS = _spmm_call(Ps, Pt, srcM, dstM, z128)

    pad_b = jnp.full((NPAD - NS,), B, jnp.int32)
    ids_s3 = jnp.concatenate([x_s_batch.astype(jnp.int32), pad_b]).reshape(
        _NB, 1, _BR)
    ids_t3 = jnp.concatenate([x_t_batch.astype(jnp.int32), pad_b]).reshape(
        _NB, 1, _BR)
    out = _f_call(accS, ea_s, Wet[2], degcol_s, hs1, hs2, ids_s3,
                  accT, ea_t, Wes[2], degcol_t, ht1, ht2, ids_t3,
                  W1a, b1a.reshape(1, -1), W2a, b2a.reshape(1, -1),
                  W1b, b1b.reshape(1, -1), W2b, b2b.reshape(1, -1),
                  Wm1, bm1.reshape(1, -1), Wm2, bm2.reshape(1, -1))
    return out


# speedup vs baseline: 2.2073x; 1.2438x over previous
import jax
import jax.numpy as jnp
from jax import lax
from jax.experimental import pallas as pl
from jax.experimental.pallas import tpu as pltpu
from jax.experimental.pallas import tpu_sc as plsc

NS = 10000
NT = 10000
E = 320000
DE = 16
H = 128
L = 3
B = 32
NPAD = 10240
GROUPS = E // 128
NSUB = 16
GPT = (GROUPS + NSUB - 1) // NSUB
RPT = NPAD // NSUB
FPT = NPAD // NSUB
EAW = NPAD * DE
EAPT = EAW // NSUB
GPT2 = 160
GROUPS2 = GPT2 * NSUB
E2 = GROUPS2 * 128
PADIDX = NS + 100

f32 = jnp.float32

_MESH = plsc.VectorSubcoreMesh(core_axis_name="c", subcore_axis_name="s")


def _deg_body(src3, dst3, zflat, deg_s_out, deg_t_out, acc, idx_v, ones_v):
    c = lax.axis_index("c")
    s = lax.axis_index("s")
    base = s * FPT
    pltpu.sync_copy(zflat.at[pl.ds(base, FPT)], acc.at[pl.ds(base, FPT)])
    for j in range(8):
        ones_v[pl.ds(j * 16, 16)] = jnp.ones((16,), f32)
    plsc.subcore_barrier()

    def body(i, carry):
        g = i * NSUB + s

        @pl.when(g < GROUPS)
        def _():
            @pl.when(c == 0)
            def _():
                pltpu.sync_copy(dst3.at[g], idx_v)

            @pl.when(c == 1)
            def _():
                pltpu.sync_copy(src3.at[g], idx_v)

            pltpu.sync_copy(ones_v, acc.at[idx_v.at[0]], add=True)

        return carry

    lax.fori_loop(0, GPT, body, 0)
    plsc.subcore_barrier()

    @pl.when(c == 0)
    def _():
        pltpu.sync_copy(acc.at[pl.ds(base, FPT)], deg_t_out.at[pl.ds(base, FPT)])

    @pl.when(c == 1)
    def _():
        pltpu.sync_copy(acc.at[pl.ds(base, FPT)], deg_s_out.at[pl.ds(base, FPT)])


_deg_call = pl.kernel(
    _deg_body,
    out_type=(
        jax.ShapeDtypeStruct((NPAD,), f32),
        jax.ShapeDtypeStruct((NPAD,), f32),
    ),
    mesh=_MESH,
    compiler_params=pltpu.CompilerParams(needs_layout_passes=False),
    scratch_types=[
        pltpu.VMEM_SHARED((NPAD,), f32),
        pltpu.VMEM((1, 128), jnp.int32),
        pltpu.VMEM((128,), f32),
    ],
)


def _ea_body(srcM, dstM, ea_hbm, invs_flat, invt_flat, z128,
             ea_t_out, ea_s_out,
             acc, gblk, sblk, ea0, ea1, out_v, wtbl_v, sem0, sem1):
    c = lax.axis_index("c")
    s = lax.axis_index("s")
    base = s * RPT
    pltpu.sync_copy(z128.at[pl.ds(base, RPT)], acc.at[pl.ds(base, RPT)])

    @pl.when(c == 0)
    def _():
        pltpu.sync_copy(invs_flat, wtbl_v)

    @pl.when(c == 1)
    def _():
        pltpu.sync_copy(invt_flat, wtbl_v)

    z16v = jnp.zeros((16,), f32)

    def zrow(r, carry):
        for k in range(8):
            out_v[r, pl.ds(k * 16, 16)] = z16v
        return carry

    lax.fori_loop(0, 128, zrow, 0)
    plsc.subcore_barrier()

    eas = (ea0, ea1)
    esems = (sem0, sem1)

    def blk_body(i2, carry):
        base_g = s * GPT2 + i2 * 16

        @pl.when(c == 0)
        def _():
            pltpu.sync_copy(srcM.at[pl.ds(base_g, 16)], gblk)
            pltpu.sync_copy(dstM.at[pl.ds(base_g, 16)], sblk)

        @pl.when(c == 1)
        def _():
            pltpu.sync_copy(dstM.at[pl.ds(base_g, 16)], gblk)
            pltpu.sync_copy(srcM.at[pl.ds(base_g, 16)], sblk)

        @pl.when(base_g < GROUPS)
        def _():
            pltpu.async_copy(ea_hbm.at[pl.ds(base_g * 2048, 2048)], eas[0],
                             esems[0])

        for j in range(16):
            g = base_g + j
            pe = j & 1
            if j + 1 < 16:
                @pl.when(g + 1 < GROUPS)
                def _(j=j, pe=pe):
                    pltpu.async_copy(
                        ea_hbm.at[pl.ds((base_g + j + 1) * 2048, 2048)],
                        eas[1 - pe], esems[1 - pe])

            @pl.when(g < GROUPS)
            def _(j=j, pe=pe):
                pltpu.make_async_copy(
                    ea_hbm.at[pl.ds((base_g + j) * 2048, 2048)],
                    eas[pe], esems[pe]).wait()

                def sub(j2, cc):
                    iv = gblk[j, pl.ds(j2 * 16, 16)]
                    w16 = plsc.load_gather(wtbl_v, [iv])
                    for m in range(16):
                        e = j2 * 16 + m
                        out_v[e, pl.ds(0, 16)] = (
                            eas[pe][pl.ds(e * 16, 16)] * w16[m])
                    return cc

                lax.fori_loop(0, 8, sub, 0)
                pltpu.sync_copy(out_v, acc.at[sblk.at[j]], add=True)

        return carry

    lax.fori_loop(0, GPT2 // 16, blk_body, 0)
    plsc.subcore_barrier()

    @pl.when(c == 0)
    def _():
        pltpu.sync_copy(acc.at[pl.ds(base, RPT)], ea_t_out.at[pl.ds(base, RPT)])

    @pl.when(c == 1)
    def _():
        pltpu.sync_copy(acc.at[pl.ds(base, RPT)], ea_s_out.at[pl.ds(base, RPT)])


_ea_call = pl.kernel(
    _ea_body,
    out_type=(
        jax.ShapeDtypeStruct((NPAD, H), f32),
        jax.ShapeDtypeStruct((NPAD, H), f32),
    ),
    mesh=_MESH,
    compiler_params=pltpu.CompilerParams(needs_layout_passes=False),
    scratch_types=[
        pltpu.VMEM_SHARED((NPAD, H), f32),
        pltpu.VMEM((16, 128), jnp.int32),
        pltpu.VMEM((16, 128), jnp.int32),
        pltpu.VMEM((128 * DE,), f32),
        pltpu.VMEM((128 * DE,), f32),
        pltpu.VMEM((128, H), f32),
        pltpu.VMEM((NPAD,), f32),
        pltpu.SemaphoreType.DMA,
        pltpu.SemaphoreType.DMA,
    ],
)


def _spmm_body(ps, pt, srcM, dstM, z128, acc_t_out, acc_s_out,
               acc, gblk, sblk, rows0, rows1, sem0, sem1):
    c = lax.axis_index("c")
    s = lax.axis_index("s")
    base = s * RPT
    pltpu.sync_copy(z128.at[pl.ds(base, RPT)], acc.at[pl.ds(base, RPT)])
    plsc.subcore_barrier()

    rows = (rows0, rows1)
    sems = (sem0, sem1)

    def fire(j, p):
        @pl.when(c == 0)
        def _():
            pltpu.async_copy(ps.at[gblk.at[j]], rows[p], sems[p])

        @pl.when(c == 1)
        def _():
            pltpu.async_copy(pt.at[gblk.at[j]], rows[p], sems[p])

    def scat(j, p):
        @pl.when(c == 0)
        def _():
            pltpu.make_async_copy(ps.at[gblk.at[j]], rows[p], sems[p]).wait()

        @pl.when(c == 1)
        def _():
            pltpu.make_async_copy(pt.at[gblk.at[j]], rows[p], sems[p]).wait()

        pltpu.sync_copy(rows[p], acc.at[sblk.at[j]], add=True)

    def blk_body(i2, carry):
        base_g = s * GPT2 + i2 * 16

        @pl.when(c == 0)
        def _():
            pltpu.sync_copy(srcM.at[pl.ds(base_g, 16)], gblk)
            pltpu.sync_copy(dstM.at[pl.ds(base_g, 16)], sblk)

        @pl.when(c == 1)
        def _():
            pltpu.sync_copy(dstM.at[pl.ds(base_g, 16)], gblk)
            pltpu.sync_copy(srcM.at[pl.ds(base_g, 16)], sblk)

        fire(0, 0)
        for j in range(16):
            p = j & 1
            if j + 1 < 16:
                fire(j + 1, 1 - p)
            scat(j, p)
        return carry

    lax.fori_loop(0, GPT2 // 16, blk_body, 0)
    plsc.subcore_barrier()

    @pl.when(c == 0)
    def _():
        pltpu.sync_copy(acc.at[pl.ds(base, RPT)], acc_t_out.at[pl.ds(base, RPT)])

    @pl.when(c == 1)
    def _():
        pltpu.sync_copy(acc.at[pl.ds(base, RPT)], acc_s_out.at[pl.ds(base, RPT)])


_spmm_call = pl.kernel(
    _spmm_body,
    out_type=(
        jax.ShapeDtypeStruct((NPAD, H), f32),
        jax.ShapeDtypeStruct((NPAD, H), f32),
    ),
    mesh=_MESH,
    compiler_params=pltpu.CompilerParams(needs_layout_passes=False),
    scratch_types=[
        pltpu.VMEM_SHARED((NPAD, H), f32),
        pltpu.VMEM((16, 128), jnp.int32),
        pltpu.VMEM((16, 128), jnp.int32),
        pltpu.VMEM((128, H), f32),
        pltpu.VMEM((128, H), f32),
        pltpu.SemaphoreType.DMA,
        pltpu.SemaphoreType.DMA,
    ],
)


_NB = 10
_BR = NPAD // _NB


def _inv_body(ds_ref, dt_ref, is_ref, it_ref):
    is_ref[...] = lax.rsqrt(jnp.maximum(ds_ref[...], 1.0))
    it_ref[...] = lax.rsqrt(jnp.maximum(dt_ref[...], 1.0))


def _inv_call(deg_s80, deg_t80):
    return pl.pallas_call(
        _inv_body,
        out_shape=(
            jax.ShapeDtypeStruct((NPAD // 128, 128), f32),
            jax.ShapeDtypeStruct((NPAD // 128, 128), f32),
        ),
    )(deg_s80, deg_t80)


def _t0_body(xs, ds, Ws0, bs0, xt, dt, Wt0, bt0, Ps, Pt):
    inv_s = lax.rsqrt(jnp.maximum(ds[...], 1.0))
    inv_t = lax.rsqrt(jnp.maximum(dt[...], 1.0))
    Ps[...] = (jnp.dot(xs[...], Ws0[...], preferred_element_type=f32)
               + bs0[...]) * inv_s
    Pt[...] = (jnp.dot(xt[...], Wt0[...], preferred_element_type=f32)
               + bt0[...]) * inv_t


def _row_spec(w):
    return pl.BlockSpec((_BR, w), lambda i: (i, 0))


def _full_spec(r, c):
    return pl.BlockSpec((r, c), lambda i: (0, 0))


def _t0_call(x_s, degcol_s, Ws0, bs0, x_t, degcol_t, Wt0, bt0):
    return pl.pallas_call(
        _t0_body,
        grid=(_NB,),
        in_specs=[
            _row_spec(H), _row_spec(1), _full_spec(H, H), _full_spec(1, H),
            _row_spec(H), _row_spec(1), _full_spec(H, H), _full_spec(1, H),
        ],
        out_specs=[_row_spec(H), _row_spec(H)],
        out_shape=(
            jax.ShapeDtypeStruct((NPAD, H), f32),
            jax.ShapeDtypeStruct((NPAD, H), f32),
        ),
    )(x_s, degcol_s, Ws0, bs0, x_t, degcol_t, Wt0, bt0)


def _t_body(accS, eaS, WetP, Wsl, bsl, ds,
            accT, eaT, WesP, Wtl, btl, dt,
            hs_o, Ps_o, ht_o, Pt_o):
    inv_s = lax.rsqrt(jnp.maximum(ds[...], 1.0))
    inv_t = lax.rsqrt(jnp.maximum(dt[...], 1.0))
    hs = jnp.maximum(inv_s * (accS[...] + jnp.dot(
        eaS[...][:, :DE], WetP[...], preferred_element_type=f32)), 0.0)
    ht = jnp.maximum(inv_t * (accT[...] + jnp.dot(
        eaT[...][:, :DE], WesP[...], preferred_element_type=f32)), 0.0)
    hs_o[...] = hs
    ht_o[...] = ht
    Ps_o[...] = (jnp.dot(hs, Wsl[...], preferred_element_type=f32)
                 + bsl[...]) * inv_s
    Pt_o[...] = (jnp.dot(ht, Wtl[...], preferred_element_type=f32)
                 + btl[...]) * inv_t


def _t_call(accS, eaS, WetP, Wsl, bsl, degcol_s,
            accT, eaT, WesP, Wtl, btl, degcol_t):
    return pl.pallas_call(
        _t_body,
        grid=(_NB,),
        in_specs=[
            _row_spec(H), _row_spec(H), _full_spec(DE, H),
            _full_spec(H, H), _full_spec(1, H), _row_spec(1),
            _row_spec(H), _row_spec(H), _full_spec(DE, H),
            _full_spec(H, H), _full_spec(1, H), _row_spec(1),
        ],
        out_specs=[_row_spec(H), _row_spec(H), _row_spec(H), _row_spec(H)],
        out_shape=(
            jax.ShapeDtypeStruct((NPAD, H), f32),
            jax.ShapeDtypeStruct((NPAD, H), f32),
            jax.ShapeDtypeStruct((NPAD, H), f32),
            jax.ShapeDtypeStruct((NPAD, H), f32),
        ),
    )(accS, eaS, WetP, Wsl, bsl, degcol_s,
      accT, eaT, WesP, Wtl, btl, degcol_t)


def _f_body(accS, eaS, WetP, ds, hs1, hs2, ids_s,
            accT, eaT, WesP, dt, ht1, ht2, ids_t,
            W1a, b1a, W2a, b2a, W1b, b1b, W2b, b2b, Wm1, bm1, Wm2, bm2,
            out, gs_ref, gt_ref):
    i = pl.program_id(0)

    @pl.when(i == 0)
    def _():
        gs_ref[...] = jnp.zeros_like(gs_ref)
        gt_ref[...] = jnp.zeros_like(gt_ref)

    inv_s = lax.rsqrt(jnp.maximum(ds[...], 1.0))
    inv_t = lax.rsqrt(jnp.maximum(dt[...], 1.0))
    hs3 = jnp.maximum(inv_s * (accS[...] + jnp.dot(
        eaS[...][:, :DE], WetP[...], preferred_element_type=f32)), 0.0)
    ht3 = jnp.maximum(inv_t * (accT[...] + jnp.dot(
        eaT[...][:, :DE], WesP[...], preferred_element_type=f32)), 0.0)
    hcat_s = jnp.concatenate([hs1[...], hs2[...], hs3], axis=1)
    hcat_t = jnp.concatenate([ht1[...], ht2[...], ht3], axis=1)
    oh_s = (lax.broadcasted_iota(jnp.int32, (B, _BR), 0)
            == ids_s[0, 0, :][None, :]).astype(f32)
    oh_t = (lax.broadcasted_iota(jnp.int32, (B, _BR), 0)
            == ids_t[0, 0, :][None, :]).astype(f32)
    gs_ref[...] += jnp.dot(oh_s, hcat_s, preferred_element_type=f32)
    gt_ref[...] += jnp.dot(oh_t, hcat_t, preferred_element_type=f32)

    @pl.when(i == _NB - 1)
    def _():
        x = jnp.concatenate([gs_ref[...], gt_ref[...]], axis=1)
        h1 = jnp.maximum(jnp.dot(x, W1a[...], preferred_element_type=f32)
                         + b1a[...], 0.0)
        x1 = jnp.dot(h1, W2a[...], preferred_element_type=f32) + b2a[...]
        h2 = jnp.maximum(jnp.dot(x, W1b[...], preferred_element_type=f32)
                         + b1b[...], 0.0)
        x2 = jnp.dot(h2, W2b[...], preferred_element_type=f32) + b2b[...]
        x12 = jnp.concatenate([x1, x2], axis=1)
        hm = jnp.maximum(jnp.dot(x12, Wm1[...], preferred_element_type=f32)
                         + bm1[...], 0.0)
        out[...] = jnp.dot(hm, Wm2[...], preferred_element_type=f32) + bm2[...]


def _f_call(accS, eaS, WetP, degcol_s, hs1, hs2, ids_s3,
            accT, eaT, WesP, degcol_t, ht1, ht2, ids_t3,
            W1a, b1a, W2a, b2a, W1b, b1b, W2b, b2b, Wm1, bm1, Wm2, bm2):
    ids_spec = pl.BlockSpec((1, 1, _BR), lambda i: (i, 0, 0))
    return pl.pallas_call(
        _f_body,
        grid=(_NB,),
        in_specs=[
            _row_spec(H), _row_spec(H), _full_spec(DE, H), _row_spec(1),
            _row_spec(H), _row_spec(H), ids_spec,
            _row_spec(H), _row_spec(H), _full_spec(DE, H), _row_spec(1),
            _row_spec(H), _row_spec(H), ids_spec,
            _full_spec(2 * L * H, H), _full_spec(1, H),
            _full_spec(H, 5), _full_spec(1, 5),
            _full_spec(2 * L * H, H), _full_spec(1, H),
            _full_spec(H, 5), _full_spec(1, 5),
            _full_spec(10, H), _full_spec(1, H),
            _full_spec(H, 1), _full_spec(1, 1),
        ],
        out_specs=pl.BlockSpec((B, 1), lambda i: (0, 0)),
        out_shape=jax.ShapeDtypeStruct((B, 1), f32),
        scratch_shapes=[
            pltpu.VMEM((B, L * H), f32),
            pltpu.VMEM((B, L * H), f32),
        ],
    )(accS, eaS, WetP, degcol_s, hs1, hs2, ids_s3,
      accT, eaT, WesP, degcol_t, ht1, ht2, ids_t3,
      W1a, b1a, W2a, b2a, W1b, b1b, W2b, b2b, Wm1, bm1, Wm2, bm2)


def kernel(x_s, x_t, edge_attr, edge_index, x_s_batch, x_t_batch,
           Ws, Wt, Wes, Wet, bs, bt,
           W1a, b1a, W2a, b2a, W1b, b1b, W2b, b2b,
           Wm1, bm1, Wm2, bm2):
    src_i = edge_index[0].astype(jnp.int32)
    dst_i = edge_index[1].astype(jnp.int32)
    src3 = src_i.reshape(GROUPS, 1, 128)
    dst3 = dst_i.reshape(GROUPS, 1, 128)
    pad_i = NS + jnp.arange(E2 - E, dtype=jnp.int32) % (NPAD - NS)
    srcM = jnp.concatenate([src_i, pad_i]).reshape(GROUPS2, 128)
    dstM = jnp.concatenate([dst_i, pad_i]).reshape(GROUPS2, 128)
    zflat = jnp.zeros((NPAD,), f32)
    z128 = jnp.zeros((NPAD, H), f32)
    zrow = jnp.zeros((NPAD - NS, H), f32)
    x_sp = jnp.concatenate([x_s, zrow])
    x_tp = jnp.concatenate([x_t, zrow])

    deg_s, deg_t = _deg_call(src3, dst3, zflat)
    inv_s80, inv_t80 = _inv_call(deg_s.reshape(NPAD // 128, 128),
                                 deg_t.reshape(NPAD // 128, 128))
    invs_flat = inv_s80.reshape(NPAD)
    invt_flat = inv_t80.reshape(NPAD)
    degcol_s = deg_s.reshape(NPAD, 1)
    degcol_t = deg_t.reshape(NPAD, 1)

    ea_t, ea_s = _ea_call(srcM, dstM, edge_attr.reshape(E * DE),
                          invs_flat, invt_flat, z128)

    bs_r = bs.reshape(L, 1, H)
    bt_r = bt.reshape(L, 1, H)

    Ps, Pt = _t0_call(x_sp, degcol_s, Ws[0], bs_r[0], x_tp, degcol_t, Wt[0], bt_r[0])
    accT, accS = _spmm_call(Ps, Pt, srcM, dstM, z128)
    hs1, Ps, ht1, Pt = _t_call(accS, ea_s, Wet[0], Ws[1], bs_r[1], degcol_s,
                               accT, ea_t, Wes[0], Wt[1], bt_r[1], degcol_t)
    accT, accS = _spmm_call(Ps, Pt, srcM, dstM, z128)
    hs2, Ps, ht2, Pt = _t_call(accS, ea_s, Wet[1], Ws[2], bs_r[2], degcol_s,
                               accT, ea_t, Wes[1], Wt[2], bt_r[2], degcol_t)
    accT, accS = _spmm_call(Ps, Pt, srcM, dstM, z128)

    pad_b = jnp.full((NPAD - NS,), B, jnp.int32)
    ids_s3 = jnp.concatenate([x_s_batch.astype(jnp.int32), pad_b]).reshape(
        _NB, 1, _BR)
    ids_t3 = jnp.concatenate([x_t_batch.astype(jnp.int32), pad_b]).reshape(
        _NB, 1, _BR)
    out = _f_call(accS, ea_s, Wet[2], degcol_s, hs1, hs2, ids_s3,
                  accT, ea_t, Wes[2], degcol_t, ht1, ht2, ids_t3,
                  W1a, b1a.reshape(1, -1), W2a, b2a.reshape(1, -1),
                  W1b, b1b.reshape(1, -1), W2b, b2b.reshape(1, -1),
                  Wm1, bm1.reshape(1, -1), Wm2, bm2.reshape(1, -1))
    return out
